# baseline (device time: 102600 ns/iter reference)
import jax
import jax.numpy as jnp
from jax import lax
from jax.experimental import pallas as pl
from jax.experimental.pallas import tpu as pltpu

N_Z = 4
N_Q = 4


def _q_to_xy(qq):
    xq = lax.div(qq, 2)
    yq = jnp.bitwise_xor(xq, lax.rem(qq, 2))
    return xq, yq


def kernel(O, Wo):
    B, S, H, D = O.shape
    K = H * D
    N = Wo.shape[1]
    s_out = S // N_Z
    n_strip = N // N_Q

    OT = jnp.transpose(O.reshape(B, S, K), (0, 2, 1))

    x_idx = lax.axis_index("x")
    y_idx = lax.axis_index("y")
    q_out = 2 * x_idx + jnp.bitwise_xor(x_idx, y_idx)
    Wq = lax.dynamic_slice(Wo, (0, q_out * n_strip), (K, n_strip))

    def body(o_ref, w_ref, out_ref, comm_ref, blocks_ref,
             p1_send, p1_recv, credit_sem,
             s1r_send, s1r_recv, s2r_send, s2r_recv,
             s1l_send, s1l_recv, s2l_send, s2l_recv, out_sem):
        my_x = lax.axis_index("x")
        my_y = lax.axis_index("y")
        my_z = lax.axis_index("z")
        q = 2 * my_x + jnp.bitwise_xor(my_x, my_y)

        zr = (my_x, my_y, lax.rem(my_z + 1, N_Z))
        zl = (my_x, my_y, lax.rem(my_z + N_Z - 1, N_Z))
        qr_x, qr_y = _q_to_xy(lax.rem(q + 1, N_Q))
        ql_x, ql_y = _q_to_xy(lax.rem(q + N_Q - 1, N_Q))
        qr = (qr_x, qr_y, my_z)
        ql = (ql_x, ql_y, my_z)

        barrier_sem = pltpu.get_barrier_semaphore()
        for nbr in (zl, zr, ql, qr):
            pl.semaphore_signal(
                barrier_sem, inc=1,
                device_id=nbr, device_id_type=pl.DeviceIdType.MESH,
            )
        pl.semaphore_wait(barrier_sem, 4)

        def partial_q(c, k):
            o = o_ref[pl.ds(k, 1), :, pl.ds(c * s_out, s_out)]
            return lax.dot_general(
                o, w_ref[:, :],
                dimension_numbers=(((1,), (0,)), ((), ())),
                preferred_element_type=jnp.float32,
            )

        def p1_copy(h, k):
            return pltpu.make_async_remote_copy(
                src_ref=comm_ref.at[h % 2, pl.ds(k, 1)],
                dst_ref=comm_ref.at[(h + 1) % 2, pl.ds(k, 1)],
                send_sem=p1_send.at[h * B + k],
                recv_sem=p1_recv.at[h * B + k],
                device_id=zr,
                device_id_type=pl.DeviceIdType.MESH,
            )

        def p2_copy(s, k, dev, sems_s, sems_r, idx):
            return pltpu.make_async_remote_copy(
                src_ref=blocks_ref.at[s, pl.ds(k, 1)],
                dst_ref=blocks_ref.at[s, pl.ds(k, 1)],
                send_sem=sems_s.at[idx],
                recv_sem=sems_r.at[idx],
                device_id=dev,
                device_id_type=pl.DeviceIdType.MESH,
            )

        c0 = lax.rem(my_z + N_Z - 1, N_Z)
        c1 = lax.rem(my_z + N_Z - 2, N_Z)
        c2 = lax.rem(my_z + 1, N_Z)

        h0 = [p1_copy(0, k) for k in range(B)]
        h1 = [p1_copy(1, k) for k in range(B)]
        h2 = [p1_copy(2, k) for k in range(B)]

        for k in range(B):
            comm_ref[0, pl.ds(k, 1)] = partial_q(c0, k)
            h0[k].start()

        for k in range(B):
            p = partial_q(c1, k)
            h0[k].wait_recv()
            comm_ref[1, pl.ds(k, 1)] = comm_ref[1, pl.ds(k, 1)] + p
            h0[k].wait_send()
            h1[k].start()

        for k in range(B):
            p = partial_q(c2, k)
            h1[k].wait_recv()
            comm_ref[0, pl.ds(k, 1)] = comm_ref[0, pl.ds(k, 1)] + p
            h1[k].wait_send()
            pl.semaphore_signal(
                credit_sem, inc=1,
                device_id=zl, device_id_type=pl.DeviceIdType.MESH,
            )
            pl.semaphore_wait(credit_sem, 1)
            h2[k].start()

        sA = lax.rem(q + N_Q - 1, N_Q)
        sB = lax.rem(q + 1, N_Q)
        r_s1 = [p2_copy(q, k, qr, s1r_send, s1r_recv, k) for k in range(B)]
        l_s1 = [p2_copy(q, k, ql, s1l_send, s1l_recv, k) for k in range(B)]
        r_s2 = [p2_copy(sA, j, qr, s2r_send, s2r_recv, j) for j in range(2)]
        l_s2 = [p2_copy(sB, j, ql, s2l_send, s2l_recv, j - 2) for j in range(2, 4)]

        for k in range(B):
            p = partial_q(my_z, k)
            h2[k].wait_recv()
            blocks_ref[q, pl.ds(k, 1)] = comm_ref[1, pl.ds(k, 1)] + p
            r_s1[k].start()
            l_s1[k].start()

        r_s1[0].wait_recv()
        r_s2[0].start()
        r_s1[1].wait_recv()
        r_s2[1].start()
        l_s1[2].wait_recv()
        l_s2[0].start()
        l_s1[3].wait_recv()
        l_s2[1].start()

        r_s1[2].wait_recv()
        r_s1[3].wait_recv()
        l_s1[0].wait_recv()
        l_s1[1].wait_recv()
        for j in range(2):
            r_s2[j].wait_recv()
            l_s2[j].wait_recv()

        out_cps = [
            pltpu.make_async_copy(
                blocks_ref.at[o],
                out_ref.at[:, :, pl.ds(o * n_strip, n_strip)],
                out_sem,
            )
            for o in range(N_Q)
        ]
        for cp in out_cps:
            cp.start()

        for k in range(B):
            h2[k].wait_send()
            r_s1[k].wait_send()
            l_s1[k].wait_send()
        for j in range(2):
            r_s2[j].wait_send()
            l_s2[j].wait_send()
        for cp in out_cps:
            cp.wait()

    return pl.pallas_call(
        body,
        out_shape=jax.ShapeDtypeStruct((B, s_out, N), jnp.float32),
        in_specs=[
            pl.BlockSpec(memory_space=pltpu.VMEM),
            pl.BlockSpec(memory_space=pltpu.VMEM),
        ],
        out_specs=pl.BlockSpec(memory_space=pl.ANY),
        scratch_shapes=[
            pltpu.VMEM((2, B, s_out, n_strip), jnp.float32),
            pltpu.VMEM((N_Q, B, s_out, n_strip), jnp.float32),
            pltpu.SemaphoreType.DMA(((N_Z - 1) * B,)),
            pltpu.SemaphoreType.DMA(((N_Z - 1) * B,)),
            pltpu.SemaphoreType.REGULAR,
            pltpu.SemaphoreType.DMA((B,)),
            pltpu.SemaphoreType.DMA((B,)),
            pltpu.SemaphoreType.DMA((2,)),
            pltpu.SemaphoreType.DMA((2,)),
            pltpu.SemaphoreType.DMA((B,)),
            pltpu.SemaphoreType.DMA((B,)),
            pltpu.SemaphoreType.DMA((2,)),
            pltpu.SemaphoreType.DMA((2,)),
            pltpu.SemaphoreType.DMA,
        ],
        compiler_params=pltpu.CompilerParams(collective_id=0),
    )(OT, Wq)


# device time: 97747 ns/iter; 1.0496x vs baseline; 1.0496x over previous
import jax
import jax.numpy as jnp
from jax import lax
from jax.experimental import pallas as pl
from jax.experimental.pallas import tpu as pltpu

N_Z = 4
N_Q = 4


def _q_to_xy(qq):
    xq = lax.div(qq, 2)
    yq = jnp.bitwise_xor(xq, lax.rem(qq, 2))
    return xq, yq


def kernel(O, Wo):
    B, S, H, D = O.shape
    K = H * D
    N = Wo.shape[1]
    s_out = S // N_Z
    n_strip = N // N_Q

    OT = jnp.transpose(O.reshape(B, S, K), (0, 2, 1))

    x_idx = lax.axis_index("x")
    y_idx = lax.axis_index("y")
    q_out = 2 * x_idx + jnp.bitwise_xor(x_idx, y_idx)
    Wq = lax.dynamic_slice(Wo, (0, q_out * n_strip), (K, n_strip))

    def body(o_ref, w_ref, out_ref, comm_ref, blocks_ref,
             p1_send, p1_recv, credit_sem,
             s1r_send, s1r_recv, s2r_send, s2r_recv,
             s1l_send, s1l_recv, s2l_send, s2l_recv):
        my_x = lax.axis_index("x")
        my_y = lax.axis_index("y")
        my_z = lax.axis_index("z")
        q = 2 * my_x + jnp.bitwise_xor(my_x, my_y)

        zr = (my_x, my_y, lax.rem(my_z + 1, N_Z))
        zl = (my_x, my_y, lax.rem(my_z + N_Z - 1, N_Z))
        qr_x, qr_y = _q_to_xy(lax.rem(q + 1, N_Q))
        ql_x, ql_y = _q_to_xy(lax.rem(q + N_Q - 1, N_Q))
        qr = (qr_x, qr_y, my_z)
        ql = (ql_x, ql_y, my_z)

        barrier_sem = pltpu.get_barrier_semaphore()
        for nbr in (zl, zr, ql, qr):
            pl.semaphore_signal(
                barrier_sem, inc=1,
                device_id=nbr, device_id_type=pl.DeviceIdType.MESH,
            )
        pl.semaphore_wait(barrier_sem, 4)

        def partial_q(c, k):
            o = o_ref[pl.ds(k, 1), :, pl.ds(c * s_out, s_out)]
            return lax.dot_general(
                o, w_ref[:, :],
                dimension_numbers=(((1,), (0,)), ((), ())),
                preferred_element_type=jnp.float32,
            )

        sh = s_out // 2

        def p1_copy(h, k):
            return pltpu.make_async_remote_copy(
                src_ref=comm_ref.at[h % 2, pl.ds(k, 1)],
                dst_ref=comm_ref.at[(h + 1) % 2, pl.ds(k, 1)],
                send_sem=p1_send.at[h * B + k],
                recv_sem=p1_recv.at[h * B + k],
                device_id=zr,
                device_id_type=pl.DeviceIdType.MESH,
            )

        def p1_copy_h2(k, j):
            return pltpu.make_async_remote_copy(
                src_ref=comm_ref.at[0, pl.ds(k, 1), pl.ds(j * sh, sh)],
                dst_ref=comm_ref.at[1, pl.ds(k, 1), pl.ds(j * sh, sh)],
                send_sem=p1_send.at[2 * B + 2 * k + j],
                recv_sem=p1_recv.at[2 * B + 2 * k + j],
                device_id=zr,
                device_id_type=pl.DeviceIdType.MESH,
            )

        def p2_copy(s, k, j, dev, sems_s, sems_r, idx):
            return pltpu.make_async_remote_copy(
                src_ref=blocks_ref.at[s, pl.ds(k, 1), pl.ds(j * sh, sh)],
                dst_ref=blocks_ref.at[s, pl.ds(k, 1), pl.ds(j * sh, sh)],
                send_sem=sems_s.at[idx],
                recv_sem=sems_r.at[idx],
                device_id=dev,
                device_id_type=pl.DeviceIdType.MESH,
            )

        c0 = lax.rem(my_z + N_Z - 1, N_Z)
        c1 = lax.rem(my_z + N_Z - 2, N_Z)
        c2 = lax.rem(my_z + 1, N_Z)

        h0 = [p1_copy(0, k) for k in range(B)]
        h1 = [p1_copy(1, k) for k in range(B)]
        h2 = [[p1_copy_h2(k, j) for j in range(2)] for k in range(B)]

        for k in range(B):
            comm_ref[0, pl.ds(k, 1)] = partial_q(c0, k)
            h0[k].start()

        for k in range(B):
            p = partial_q(c1, k)
            h0[k].wait_recv()
            comm_ref[1, pl.ds(k, 1)] = comm_ref[1, pl.ds(k, 1)] + p
            h0[k].wait_send()
            h1[k].start()

        for k in range(B):
            p = partial_q(c2, k)
            h1[k].wait_recv()
            comm_ref[0, pl.ds(k, 1)] = comm_ref[0, pl.ds(k, 1)] + p
            h1[k].wait_send()
            pl.semaphore_signal(
                credit_sem, inc=1,
                device_id=zl, device_id_type=pl.DeviceIdType.MESH,
            )
            pl.semaphore_wait(credit_sem, 1)
            h2[k][0].start()
            h2[k][1].start()

        sA = lax.rem(q + N_Q - 1, N_Q)
        sB = lax.rem(q + 1, N_Q)
        r_s1 = [[p2_copy(q, k, j, qr, s1r_send, s1r_recv, 2 * k + j)
                 for j in range(2)] for k in range(B)]
        l_s1 = [[p2_copy(q, k, j, ql, s1l_send, s1l_recv, 2 * k + j)
                 for j in range(2)] for k in range(B)]
        r_s2 = [[p2_copy(sA, kq, j, qr, s2r_send, s2r_recv, 2 * kq + j)
                 for j in range(2)] for kq in range(2)]
        l_s2 = [[p2_copy(sB, kq, j, ql, s2l_send, s2l_recv, 2 * (kq - 2) + j)
                 for j in range(2)] for kq in range(2, 4)]

        for k in range(B):
            p = partial_q(my_z, k)
            for j in range(2):
                h2[k][j].wait_recv()
                blocks_ref[q, pl.ds(k, 1), pl.ds(j * sh, sh)] = (
                    comm_ref[1, pl.ds(k, 1), pl.ds(j * sh, sh)]
                    + p[:, j * sh:(j + 1) * sh, :]
                )
                r_s1[k][j].start()
                l_s1[k][j].start()

        for j in range(2):
            r_s1[0][j].wait_recv()
            r_s2[0][j].start()
        for j in range(2):
            r_s1[1][j].wait_recv()
            r_s2[1][j].start()
        for j in range(2):
            l_s1[2][j].wait_recv()
            l_s2[0][j].start()
        for j in range(2):
            l_s1[3][j].wait_recv()
            l_s2[1][j].start()

        for k in (2, 3):
            for j in range(2):
                r_s1[k][j].wait_recv()
        for k in (0, 1):
            for j in range(2):
                l_s1[k][j].wait_recv()
        for kq in range(2):
            for j in range(2):
                r_s2[kq][j].wait_recv()
                l_s2[kq][j].wait_recv()

        for o in range(N_Q):
            out_ref[:, :, o * n_strip:(o + 1) * n_strip] = blocks_ref[o]

        for k in range(B):
            for j in range(2):
                h2[k][j].wait_send()
                r_s1[k][j].wait_send()
                l_s1[k][j].wait_send()
        for kq in range(2):
            for j in range(2):
                r_s2[kq][j].wait_send()
                l_s2[kq][j].wait_send()

    return pl.pallas_call(
        body,
        out_shape=jax.ShapeDtypeStruct((B, s_out, N), jnp.float32),
        in_specs=[
            pl.BlockSpec(memory_space=pltpu.VMEM),
            pl.BlockSpec(memory_space=pltpu.VMEM),
        ],
        out_specs=pl.BlockSpec(memory_space=pltpu.VMEM),
        scratch_shapes=[
            pltpu.VMEM((2, B, s_out, n_strip), jnp.float32),
            pltpu.VMEM((N_Q, B, s_out, n_strip), jnp.float32),
            pltpu.SemaphoreType.DMA((4 * B,)),
            pltpu.SemaphoreType.DMA((4 * B,)),
            pltpu.SemaphoreType.REGULAR,
            pltpu.SemaphoreType.DMA((2 * B,)),
            pltpu.SemaphoreType.DMA((2 * B,)),
            pltpu.SemaphoreType.DMA((4,)),
            pltpu.SemaphoreType.DMA((4,)),
            pltpu.SemaphoreType.DMA((2 * B,)),
            pltpu.SemaphoreType.DMA((2 * B,)),
            pltpu.SemaphoreType.DMA((4,)),
            pltpu.SemaphoreType.DMA((4,)),
        ],
        compiler_params=pltpu.CompilerParams(collective_id=0),
    )(OT, Wq)
